# baseline (speedup 1.0000x reference)
import functools

import jax
import jax.numpy as jnp
from jax import lax
from jax.experimental import pallas as pl
from jax.experimental.pallas import tpu as pltpu
from jax.experimental.pallas import tpu_sc as plsc

_B = 16384
_L = 200
_D = 32

_info = plsc.get_sparse_core_info()
_NC, _NS = _info.num_cores, _info.num_subcores
_NW = _NC * _NS
_PER_W = _B // _NW
_NBUF = 8
_NGROUP = _PER_W // _NBUF

_mesh = plsc.VectorSubcoreMesh(core_axis_name="c", subcore_axis_name="s")


@functools.partial(
    pl.kernel,
    mesh=_mesh,
    out_type=jax.ShapeDtypeStruct((_B, _L, _D), jnp.float32),
    scratch_types=[
        pltpu.VMEM((_NBUF, _L), jnp.int32),
        pltpu.VMEM((_NBUF, _L, _D), jnp.float32),
        pltpu.SemaphoreType.DMA,
        pltpu.SemaphoreType.DMA,
        pltpu.SemaphoreType.DMA,
    ],
    compiler_params=pltpu.CompilerParams(use_tc_tiling_on_sc=False),
)
def _gather_all(actions_hbm, table_hbm, out_hbm, idx_v, rows_v, isem, gsem, osem):
    wid = lax.axis_index("s") * _NC + lax.axis_index("c")
    base = wid * _PER_W

    def idx_cp(c, b):
        return pltpu.make_async_copy(actions_hbm.at[base + c], idx_v.at[b], isem)

    def gat_cp(b):
        return pltpu.make_async_copy(table_hbm.at[idx_v.at[b]], rows_v.at[b], gsem)

    def out_cp(c, b):
        return pltpu.make_async_copy(rows_v.at[b], out_hbm.at[base + c], osem)

    for b in range(_NBUF):
        idx_cp(b, b).start()
    idx_cp(0, 0).wait()
    gat_cp(0).start()

    def body(g, carry):
        for j in range(_NBUF):
            c = g * _NBUF + j
            b = j
            bp = (j - 1) % _NBUF

            @pl.when(c > 0)
            def _():
                idx_cp(c, b).wait()

                @pl.when(c >= _NBUF)
                def _():
                    out_cp(c - _NBUF, b).wait()

                gat_cp(b).start()

            @pl.when(c > 0)
            def _():
                gat_cp(bp).wait()
                out_cp(c - 1, bp).start()

                @pl.when(c - 1 + _NBUF < _PER_W)
                def _():
                    idx_cp(c - 1 + _NBUF, bp).start()

        return carry

    lax.fori_loop(0, _NGROUP, body, 0)

    last_b = (_PER_W - 1) % _NBUF
    gat_cp(last_b).wait()
    out_cp(_PER_W - 1, last_b).start()
    for k in range(_NBUF):
        c = _PER_W - _NBUF + k
        out_cp(c, c % _NBUF).wait()


def kernel(actions, table):
    return _gather_all(actions.astype(jnp.int32), table)
